# baseline (speedup 1.0000x reference)
import functools

import jax
import jax.numpy as jnp
from jax import lax
from jax.experimental import pallas as pl
from jax.experimental.pallas import tpu as pltpu
from jax.experimental.pallas import tpu_sc as plsc

BATCH = 16384
HIST = 200
EMBED_DIM = 64
LANES = 16
BB = 128
NUM_BLOCKS = BATCH // BB
NUM_WORKERS = 32
BLOCKS_PER_W = NUM_BLOCKS // NUM_WORKERS
NUM_PAIRS = HIST // 2


def _gather_transposed(table, idx_t):
    out_shape = (HIST * 8, NUM_BLOCKS, 8 * BB)

    @functools.partial(
        pl.kernel,
        out_type=jax.ShapeDtypeStruct(out_shape, jnp.float32),
        mesh=plsc.VectorSubcoreMesh(
            core_axis_name="c", subcore_axis_name="s"
        ),
        scratch_types=[
            pltpu.VMEM((HIST, BB), jnp.int32),
            pltpu.VMEM((BB, EMBED_DIM), jnp.float32),
            pltpu.VMEM((BB, EMBED_DIM), jnp.float32),
            pltpu.VMEM((8, 8 * BB), jnp.float32),
            pltpu.VMEM((8, 8 * BB), jnp.float32),
            pltpu.SemaphoreType.DMA,
            pltpu.SemaphoreType.DMA,
            pltpu.SemaphoreType.DMA,
            pltpu.SemaphoreType.DMA,
        ],
        compiler_params=pltpu.CompilerParams(
            use_tc_tiling_on_sc=False, needs_layout_passes=False
        ),
    )
    def k(table_hbm, idx_hbm, out_hbm,
          idxb, gbuf0, gbuf1, tbuf0, tbuf1, gsem0, gsem1, ssem0, ssem1):
        wid = lax.axis_index("s") * 2 + lax.axis_index("c")

        def fire_gather(h, gbuf, sem):
            pltpu.async_copy(table_hbm.at[idxb.at[h]], gbuf, sem)

        def gather_wait(h, gbuf, sem):
            pltpu.make_async_copy(
                table_hbm.at[idxb.at[h]], gbuf, sem
            ).wait()

        def fire_store(h, bt, tbuf, sem):
            pltpu.async_copy(
                tbuf, out_hbm.at[pl.ds(h * 8, 8), bt], sem
            )

        def store_wait(h, bt, tbuf, sem):
            pltpu.make_async_copy(
                tbuf, out_hbm.at[pl.ds(h * 8, 8), bt], sem
            ).wait()

        rowv = [
            lax.iota(jnp.int32, LANES) + (bj * LANES) for bj in range(8)
        ]

        def transpose(gbuf, tbuf):
            def dt_body(dt, _):
                for di in range(8):
                    d = dt * 8 + di
                    colv = jnp.full((LANES,), 0, jnp.int32) + d
                    for bj in range(8):
                        v = plsc.load_gather(gbuf, [rowv[bj], colv])
                        tbuf[dt, pl.ds(di * BB + bj * LANES, LANES)] = v
                return 0
            lax.fori_loop(0, 8, dt_body, 0)

        def block_body(blk, _):
            bt = wid * BLOCKS_PER_W + blk
            pltpu.sync_copy(
                idx_hbm.at[pl.ds(0, HIST), pl.ds(bt * BB, BB)], idxb
            )
            fire_gather(0, gbuf0, gsem0)

            def pair_body(p, _):
                h0 = 2 * p
                h1 = h0 + 1

                fire_gather(h1, gbuf1, gsem1)
                gather_wait(h0, gbuf0, gsem0)

                @pl.when(p > 0)
                def _():
                    store_wait(h0 - 2, bt, tbuf0, ssem0)

                transpose(gbuf0, tbuf0)
                fire_store(h0, bt, tbuf0, ssem0)

                @pl.when(p < NUM_PAIRS - 1)
                def _():
                    fire_gather(h0 + 2, gbuf0, gsem0)

                gather_wait(h1, gbuf1, gsem1)

                @pl.when(p > 0)
                def _():
                    store_wait(h1 - 2, bt, tbuf1, ssem1)

                transpose(gbuf1, tbuf1)
                fire_store(h1, bt, tbuf1, ssem1)
                return 0

            lax.fori_loop(0, NUM_PAIRS, pair_body, 0)

            store_wait(HIST - 2, bt, tbuf0, ssem0)
            store_wait(HIST - 1, bt, tbuf1, ssem1)
            return 0

        lax.fori_loop(0, BLOCKS_PER_W, block_body, 0)

    return k(table, idx_t)


def kernel(x, table):
    idx_t = x.T.astype(jnp.int32)
    out3 = _gather_transposed(table, idx_t)
    out6 = out3.reshape(HIST, 8, NUM_BLOCKS, 8, BB)
    return jnp.transpose(out6, (2, 4, 0, 1, 3)).reshape(BATCH, HIST, EMBED_DIM)
